# baseline (speedup 1.0000x reference)
import functools

import jax
import jax.numpy as jnp
from jax import lax
from jax.experimental import pallas as pl
from jax.experimental.pallas import tpu as pltpu
from jax.experimental.pallas import tpu_sc as plsc

B = 16384
HIST = 50
D = 64
H = 64

NC = 2
NS = 16
NW = NC * NS
ROWS_W = B // NW
CPW = ROWS_W // 2
IPC = 112


def _pooled_sums(x2, emb):
    mesh = plsc.VectorSubcoreMesh(
        core_axis_name="c", subcore_axis_name="s", num_cores=NC, num_subcores=NS
    )

    @functools.partial(
        pl.kernel,
        out_type=jax.ShapeDtypeStruct((B, D), jnp.float32),
        mesh=mesh,
        compiler_params=pltpu.CompilerParams(use_tc_tiling_on_sc=False),
        scratch_types=[
            pltpu.VMEM((CPW, IPC), jnp.int32),
            pltpu.VMEM((IPC,), jnp.int32),
            pltpu.VMEM((IPC,), jnp.int32),
            pltpu.VMEM((IPC, D), jnp.float32),
            pltpu.VMEM((IPC, D), jnp.float32),
            pltpu.VMEM((ROWS_W, D), jnp.float32),
            pltpu.SemaphoreType.DMA,
            pltpu.SemaphoreType.DMA,
        ],
    )
    def pool(x2_hbm, emb_hbm, out_hbm, idx_all, ib0, ib1, gbuf0, gbuf1, out_v,
             sem0, sem1):
        wid = lax.axis_index("s") * NC + lax.axis_index("c")
        cbase = wid * CPW

        pltpu.sync_copy(x2_hbm.at[pl.ds(cbase, CPW)], idx_all)

        gbufs = (gbuf0, gbuf1)
        ibufs = (ib0, ib1)
        sems = (sem0, sem1)

        def load_idx(j, ib):
            for c in range(IPC // 16):
                ib[pl.ds(c * 16, 16)] = idx_all[j, pl.ds(c * 16, 16)]

        load_idx(0, ib0)
        pltpu.async_copy(emb_hbm.at[ib0], gbuf0, sem0)

        def body(jj, carry):
            for b in range(2):
                j = jj * 2 + b

                @pl.when(j + 1 < CPW)
                def _start_next():
                    load_idx(j + 1, ibufs[1 - b])
                    pltpu.async_copy(
                        emb_hbm.at[ibufs[1 - b]], gbufs[1 - b], sems[1 - b]
                    )

                gb = gbufs[b]
                pltpu.make_async_copy(emb_hbm.at[ibufs[b]], gb, sems[b]).wait()

                for r in range(2):
                    acc = [gb[r * HIST, pl.ds(c * 16, 16)] for c in range(4)]
                    for t in range(1, HIST):
                        for c in range(4):
                            acc[c] = acc[c] + gb[r * HIST + t, pl.ds(c * 16, 16)]
                    row = j * 2 + r
                    for c in range(4):
                        out_v[row, pl.ds(c * 16, 16)] = acc[c]
            return carry

        lax.fori_loop(0, CPW // 2, body, 0)

        pltpu.sync_copy(out_v, out_hbm.at[pl.ds(wid * ROWS_W, ROWS_W)])

    return pool(x2, emb)


BT = 2048


def _mlp_body(p_ref, w1_ref, b1_ref, w2_ref, b2_ref, o_ref):
    p = p_ref[...] * (1.0 / HIST)
    h = lax.dot_general(
        p, w1_ref[...], (((1,), (1,)), ((), ())), preferred_element_type=jnp.float32
    )
    h = jnp.maximum(h + b1_ref[...], 0.0)
    o = jnp.sum(h * w2_ref[...], axis=1, keepdims=True)
    o_ref[...] = o + b2_ref[0, 0]


def _mlp(pooled, W1, b1, W2, b2):
    return pl.pallas_call(
        _mlp_body,
        grid=(B // BT,),
        in_specs=[
            pl.BlockSpec((BT, D), lambda i: (i, 0)),
            pl.BlockSpec((H, D), lambda i: (0, 0)),
            pl.BlockSpec((1, H), lambda i: (0, 0)),
            pl.BlockSpec((1, H), lambda i: (0, 0)),
            pl.BlockSpec(memory_space=pltpu.SMEM),
        ],
        out_specs=pl.BlockSpec((BT, 1), lambda i: (i, 0)),
        out_shape=jax.ShapeDtypeStruct((B, 1), jnp.float32),
    )(pooled, W1, b1.reshape(1, H), W2, b2.reshape(1, 1))


def kernel(x, emb, W1, b1, W2, b2):
    x2 = jnp.pad(
        x.astype(jnp.int32).reshape(B // 2, 2 * HIST), ((0, 0), (0, IPC - 2 * HIST))
    )
    pooled = _pooled_sums(x2, emb)
    return _mlp(pooled, W1, b1, W2, b2)


# speedup vs baseline: 1.0089x; 1.0089x over previous
import functools

import jax
import jax.numpy as jnp
from jax import lax
from jax.experimental import pallas as pl
from jax.experimental.pallas import tpu as pltpu
from jax.experimental.pallas import tpu_sc as plsc

B = 16384
HIST = 50
D = 64
H = 64

NC = 2
NS = 16
NW = NC * NS
ROWS_W = B // NW
CPW = ROWS_W // 2
IPC = 112
NB = 8


def _pooled_sums(x2, emb):
    mesh = plsc.VectorSubcoreMesh(
        core_axis_name="c", subcore_axis_name="s", num_cores=NC, num_subcores=NS
    )

    @functools.partial(
        pl.kernel,
        out_type=jax.ShapeDtypeStruct((B, D), jnp.float32),
        mesh=mesh,
        compiler_params=pltpu.CompilerParams(use_tc_tiling_on_sc=False),
        scratch_types=[
            pltpu.VMEM((CPW, IPC), jnp.int32),
            [pltpu.VMEM((IPC,), jnp.int32) for _ in range(NB)],
            [pltpu.VMEM((IPC, D), jnp.float32) for _ in range(NB)],
            pltpu.VMEM((ROWS_W, D), jnp.float32),
            [pltpu.SemaphoreType.DMA for _ in range(NB)],
        ],
    )
    def pool(x2_hbm, emb_hbm, out_hbm, idx_all, ibufs, gbufs, out_v, sems):
        wid = lax.axis_index("s") * NC + lax.axis_index("c")
        cbase = wid * CPW

        pltpu.sync_copy(x2_hbm.at[pl.ds(cbase, CPW)], idx_all)

        def load_idx(j, ib):
            for c in range(IPC // 16):
                ib[pl.ds(c * 16, 16)] = idx_all[j, pl.ds(c * 16, 16)]

        for p in range(NB - 1):
            load_idx(p, ibufs[p])
            pltpu.async_copy(emb_hbm.at[ibufs[p]], gbufs[p], sems[p])

        def body(jj, carry):
            for b in range(NB):
                j = jj * NB + b
                nxt = j + NB - 1

                @pl.when(nxt < CPW)
                def _start_next():
                    s = (b + NB - 1) % NB
                    load_idx(nxt, ibufs[s])
                    pltpu.async_copy(emb_hbm.at[ibufs[s]], gbufs[s], sems[s])

                gb = gbufs[b]
                pltpu.make_async_copy(emb_hbm.at[ibufs[b]], gb, sems[b]).wait()

                for r in range(2):
                    zero = jnp.zeros((16,), jnp.float32)

                    def red_body(k, accs, _r=r, _gb=gb):
                        base = _r * HIST + k * 10
                        out = list(accs)
                        for t in range(10):
                            for c in range(4):
                                out[c] = out[c] + _gb[base + t, pl.ds(c * 16, 16)]
                        return tuple(out)

                    accs = lax.fori_loop(0, HIST // 10, red_body,
                                         (zero, zero, zero, zero))
                    row = j * 2 + r
                    for c in range(4):
                        out_v[row, pl.ds(c * 16, 16)] = accs[c]
            return carry

        lax.fori_loop(0, CPW // NB, body, 0)

        pltpu.sync_copy(out_v, out_hbm.at[pl.ds(wid * ROWS_W, ROWS_W)])

    return pool(x2, emb)


BT = 2048


def _mlp_body(p_ref, w1_ref, b1_ref, w2_ref, b2_ref, o_ref):
    p = p_ref[...] * (1.0 / HIST)
    h = lax.dot_general(
        p, w1_ref[...], (((1,), (1,)), ((), ())), preferred_element_type=jnp.float32
    )
    h = jnp.maximum(h + b1_ref[...], 0.0)
    o = jnp.sum(h * w2_ref[...], axis=1, keepdims=True)
    o_ref[...] = o + b2_ref[0, 0]


def _mlp(pooled, W1, b1, W2, b2):
    return pl.pallas_call(
        _mlp_body,
        grid=(B // BT,),
        in_specs=[
            pl.BlockSpec((BT, D), lambda i: (i, 0)),
            pl.BlockSpec((H, D), lambda i: (0, 0)),
            pl.BlockSpec((1, H), lambda i: (0, 0)),
            pl.BlockSpec((1, H), lambda i: (0, 0)),
            pl.BlockSpec(memory_space=pltpu.SMEM),
        ],
        out_specs=pl.BlockSpec((BT, 1), lambda i: (i, 0)),
        out_shape=jax.ShapeDtypeStruct((B, 1), jnp.float32),
    )(pooled, W1, b1.reshape(1, H), W2, b2.reshape(1, 1))


def kernel(x, emb, W1, b1, W2, b2):
    x2 = jnp.pad(
        x.astype(jnp.int32).reshape(B // 2, 2 * HIST), ((0, 0), (0, IPC - 2 * HIST))
    )
    pooled = _pooled_sums(x2, emb)
    return _mlp(pooled, W1, b1, W2, b2)


# speedup vs baseline: 3.3120x; 3.2827x over previous
import functools

import jax
import jax.numpy as jnp
from jax import lax
from jax.experimental import pallas as pl
from jax.experimental.pallas import tpu as pltpu
from jax.experimental.pallas import tpu_sc as plsc

B = 16384
HIST = 50
D = 64
DP = 128
H = 64

NC = 2
NS = 16
NW = NC * NS
ROWS_W = B // NW
SPC = 8
TPC = SPC * HIST
CPW = ROWS_W // SPC
TOK_W = ROWS_W * HIST
QTOK = TOK_W // 4
NB = 2


def _pooled_sums(x1, emb2):
    mesh = plsc.VectorSubcoreMesh(
        core_axis_name="c", subcore_axis_name="s", num_cores=NC, num_subcores=NS
    )

    @functools.partial(
        pl.kernel,
        out_type=jax.ShapeDtypeStruct((B, DP), jnp.float32),
        mesh=mesh,
        compiler_params=pltpu.CompilerParams(use_tc_tiling_on_sc=True),
        scratch_types=[
            pltpu.VMEM((2 * QTOK,), jnp.int32),
            [pltpu.VMEM((TPC, DP), jnp.float32) for _ in range(NB)],
            [pltpu.VMEM((SPC, DP), jnp.float32) for _ in range(NB)],
            [pltpu.SemaphoreType.DMA for _ in range(NB)],
            pltpu.SemaphoreType.DMA,
        ],
    )
    def pool(x1_hbm, emb_hbm, out_hbm, idx_all, gbufs, outcs, sems, osem):
        wid = lax.axis_index("s") * NC + lax.axis_index("c")

        def load_quarter(q):
            pltpu.sync_copy(
                x1_hbm.at[pl.ds(wid * TOK_W + q * QTOK, QTOK)],
                idx_all.at[pl.ds(lax.rem(q, 2) * QTOK, QTOK)],
            )

        def chunk_base(j):
            return (
                lax.bitwise_and(lax.shift_right_logical(j, 4), 1) * QTOK
                + lax.bitwise_and(j, 15) * TPC
            )

        load_quarter(0)

        def start_gathers(j, gb, sem):
            base = chunk_base(j)
            for c in range(TPC // 16):
                iv = lax.shift_right_logical(
                    idx_all[pl.ds(base + c * 16, 16)], 1
                )
                pltpu.async_copy(emb_hbm.at[iv], gb.at[pl.ds(c * 16, 16)], sem)

        def wait_gathers(gb, sem):
            dummy = jnp.zeros((16,), jnp.int32)
            for c in range(TPC // 16):
                pltpu.make_async_copy(
                    emb_hbm.at[dummy], gb.at[pl.ds(c * 16, 16)], sem
                ).wait()

        for p in range(NB - 1):
            start_gathers(p, gbufs[p], sems[p])

        def body(jj, carry):
            for b in range(NB):
                j = jj * NB + b

                nxt = j + NB - 1

                @pl.when(nxt < CPW)
                def _start_next():
                    @pl.when(lax.bitwise_and(nxt, 15) == 0)
                    def _reload():
                        load_quarter(lax.shift_right_logical(nxt, 4))

                    s = (b + NB - 1) % NB
                    start_gathers(nxt, gbufs[s], sems[s])

                gb = gbufs[b]
                oc = outcs[b]
                wait_gathers(gb, sems[b])

                @pl.when(jj > 0)
                def _drain_out():
                    pltpu.make_async_copy(oc, out_hbm.at[pl.ds(0, SPC)],
                                          osem).wait()

                base = chunk_base(j)

                def sample_body(r, carry, _gb=gb, _oc=oc, _base=base):
                    sbase = _base + r * HIST
                    rbase = r * HIST
                    offs = []
                    for q, qoff in enumerate((0, 16, 32, 34)):
                        pv = lax.shift_left(
                            lax.bitwise_and(
                                idx_all[pl.ds(sbase + qoff, 16)], 1
                            ),
                            6,
                        )
                        lanes = range(16) if q < 3 else range(14, 16)
                        for lane in lanes:
                            offs.append((qoff + lane, pv[lane]))
                    acc = [jnp.zeros((16,), jnp.float32) for _ in range(4)]
                    for t, off in offs:
                        for c in range(4):
                            acc[c] = acc[c] + _gb[rbase + t,
                                                  pl.ds(off + c * 16, 16)]
                    for c in range(4):
                        _oc[r, pl.ds(c * 16, 16)] = acc[c]
                    zv = jnp.zeros((16,), jnp.float32)
                    for c in range(4, 8):
                        _oc[r, pl.ds(c * 16, 16)] = zv
                    return carry

                lax.fori_loop(0, SPC, sample_body, 0)

                pltpu.async_copy(
                    oc, out_hbm.at[pl.ds(wid * ROWS_W + j * SPC, SPC)], osem
                )
            return carry

        lax.fori_loop(0, CPW // NB, body, 0)

        for b in range(NB):
            pltpu.make_async_copy(outcs[b], out_hbm.at[pl.ds(0, SPC)],
                                  osem).wait()

    return pool(x1, emb2)


BT = 2048


def _mlp_body(p_ref, w1_ref, b1_ref, w2_ref, b2_ref, o_ref):
    p = p_ref[...][:, :D] * (1.0 / HIST)
    h = lax.dot_general(
        p, w1_ref[...], (((1,), (1,)), ((), ())), preferred_element_type=jnp.float32
    )
    h = jnp.maximum(h + b1_ref[...], 0.0)
    o = jnp.sum(h * w2_ref[...], axis=1, keepdims=True)
    o_ref[...] = o + b2_ref[0, 0]


def _mlp(pooled, W1, b1, W2, b2):
    return pl.pallas_call(
        _mlp_body,
        grid=(B // BT,),
        in_specs=[
            pl.BlockSpec((BT, DP), lambda i: (i, 0)),
            pl.BlockSpec((H, D), lambda i: (0, 0)),
            pl.BlockSpec((1, H), lambda i: (0, 0)),
            pl.BlockSpec((1, H), lambda i: (0, 0)),
            pl.BlockSpec(memory_space=pltpu.SMEM),
        ],
        out_specs=pl.BlockSpec((BT, 1), lambda i: (i, 0)),
        out_shape=jax.ShapeDtypeStruct((B, 1), jnp.float32),
    )(pooled, W1, b1.reshape(1, H), W2, b2.reshape(1, 1))


def kernel(x, emb, W1, b1, W2, b2):
    x1 = x.astype(jnp.int32).reshape(B * HIST)
    emb2 = emb.reshape(emb.shape[0] // 2, DP)
    pooled = _pooled_sums(x1, emb2)
    return _mlp(pooled, W1, b1, W2, b2)
